# Initial kernel scaffold; baseline (speedup 1.0000x reference)
import functools

import jax
import jax.numpy as jnp
from jax import lax
from jax.experimental import pallas as pl
from jax.experimental.pallas import tpu as pltpu
from jax.experimental.pallas import tpu_sc as plsc

_THRESH = 0.7
_MIN_KEPT = 100000

_NC = 2
_NS = 16
_L = 16
_NW = _NC * _NS


def _loss_body(w_ref, x_ref, t_ref, o_ref):
    x = x_ref[0]
    t = t_ref[0]
    w = w_ref[:, :1]
    m = jnp.max(x, axis=0, keepdims=True)
    s = jnp.sum(jnp.exp(x - m), axis=0, keepdims=True)
    lse = m + jnp.log(s)
    cls = lax.broadcasted_iota(jnp.int32, x.shape, 0)
    oh = cls == t
    xt = jnp.sum(jnp.where(oh, x, 0.0), axis=0, keepdims=True)
    wt = jnp.sum(jnp.where(oh, w, 0.0), axis=0, keepdims=True)
    o_ref[0] = jnp.maximum(wt * (lse - xt), 0.0)


def _compute_loss(logits, targets, weights):
    B, C, H, W = logits.shape
    N = H * W
    chunk = 8192 if N % 8192 == 0 else N
    x3 = logits.reshape(B, C, N)
    t3 = targets.reshape(B, 1, N)
    w2 = jnp.broadcast_to(weights[:, None], (C, 128))
    loss = pl.pallas_call(
        _loss_body,
        grid=(B, N // chunk),
        in_specs=[
            pl.BlockSpec((C, 128), lambda b, c: (0, 0)),
            pl.BlockSpec((1, C, chunk), lambda b, c: (b, 0, c)),
            pl.BlockSpec((1, 1, chunk), lambda b, c: (b, 0, c)),
        ],
        out_specs=pl.BlockSpec((1, 1, chunk), lambda b, c: (b, 0, c)),
        out_shape=jax.ShapeDtypeStruct((B, 1, N), jnp.float32),
    )(w2, x3, t3)
    return loss.reshape(B * N)


def _make_hist_round(n, shift, first):
    per_tile = n // _NW
    schunk = min(4096, per_tile)
    nchunks = per_tile // schunk
    niter = schunk // _L
    mesh = plsc.VectorSubcoreMesh(core_axis_name="c", subcore_axis_name="s")

    @functools.partial(
        pl.kernel,
        out_type=(
            jax.ShapeDtypeStruct((_NW, 256, _L), jnp.int32),
            jax.ShapeDtypeStruct((_NW, 256, _L), jnp.float32),
        ),
        mesh=mesh,
        scratch_types=[
            pltpu.VMEM((schunk,), jnp.float32),
            pltpu.VMEM((256, _L), jnp.int32),
            pltpu.VMEM((256, _L), jnp.float32),
            pltpu.VMEM((_L,), jnp.int32),
        ],
    )
    def round_kernel(loss_hbm, pref_hbm, cnt_hbm, sum_hbm,
                     data_v, cnt_v, sum_v, pref_v):
        cid = lax.axis_index("c")
        sid = lax.axis_index("s")
        wid = sid * _NC + cid
        base = wid * per_tile
        pltpu.sync_copy(pref_hbm, pref_v)
        prefu = plsc.bitcast(pref_v[...], jnp.uint32)
        zeros_i = jnp.zeros((_L,), jnp.int32)
        zeros_f = jnp.zeros((_L,), jnp.float32)
        ones_i = jnp.ones((_L,), jnp.int32)
        lanes = lax.iota(jnp.int32, _L)

        def zinit(b, carry):
            cnt_v[b] = zeros_i
            sum_v[b] = zeros_f
            return carry

        lax.fori_loop(0, 256, zinit, 0)

        def chunk_body(c, carry):
            pltpu.sync_copy(loss_hbm.at[pl.ds(base + c * schunk, schunk)],
                            data_v)

            def inner(i, icarry):
                v = data_v[pl.ds(i * _L, _L)]
                u = plsc.bitcast(v, jnp.uint32)
                dig = lax.shift_right_logical(u, jnp.uint32(shift))
                dig = dig & jnp.uint32(0xFF)
                bin_i = plsc.bitcast(dig, jnp.int32)
                if first:
                    pm = None
                else:
                    hi = lax.shift_right_logical(u, jnp.uint32(shift + 8))
                    pm = hi == prefu
                plsc.addupdate_scatter(cnt_v, [bin_i, lanes], ones_i, mask=pm)
                plsc.addupdate_scatter(sum_v, [bin_i, lanes], v, mask=pm)
                return icarry

            lax.fori_loop(0, niter, inner, 0)
            return carry

        lax.fori_loop(0, nchunks, chunk_body, 0)
        pltpu.sync_copy(cnt_v, cnt_hbm.at[wid])
        pltpu.sync_copy(sum_v, sum_hbm.at[wid])

    return round_kernel


def _hist_round(loss, pref_splat, shift, first):
    n = loss.shape[0]
    fn = _make_hist_round(n, shift, first)
    return fn(loss, pref_splat)


def kernel(logits, targets, weights):
    B, C, H, W = logits.shape
    n = B * H * W
    loss = _compute_loss(logits, targets, weights)
    if _MIN_KEPT >= n:
        return jnp.mean(loss)

    k = int(_THRESH * n)
    remaining = jnp.int32(k)
    s_gt = jnp.float32(0.0)
    prefix = jnp.int32(0)
    for r, shift in enumerate((24, 16, 8, 0)):
        pref_splat = jnp.broadcast_to(prefix, (_L,))
        cnt3, sum3 = _hist_round(loss, pref_splat, shift, r == 0)
        cnt = jnp.sum(cnt3, axis=(0, 2))
        sm = jnp.sum(sum3, axis=(0, 2))
        suf_c = jnp.cumsum(cnt[::-1])[::-1]
        suf_s = jnp.cumsum(sm[::-1])[::-1]
        b = jnp.sum((suf_c >= remaining).astype(jnp.int32)) - 1
        above_cnt = suf_c[b] - cnt[b]
        above_sum = suf_s[b] - sm[b]
        remaining = remaining - above_cnt
        s_gt = s_gt + above_sum
        prefix = (prefix << 8) | b
    tau = lax.bitcast_convert_type(prefix, jnp.float32)
    return (s_gt + remaining.astype(jnp.float32) * tau) / jnp.float32(k)


# speedup vs baseline: 32.7201x; 32.7201x over previous
import functools

import jax
import jax.numpy as jnp
from jax import lax
from jax.experimental import pallas as pl
from jax.experimental.pallas import tpu as pltpu
from jax.experimental.pallas import tpu_sc as plsc

_THRESH = 0.7
_MIN_KEPT = 100000

_NC = 2
_NS = 16
_L = 16
_NW = _NC * _NS


def _loss_body(w_ref, x_ref, t_ref, o_ref):
    x = x_ref[0]
    t = t_ref[0]
    w = w_ref[:, :1]
    m = jnp.max(x, axis=0, keepdims=True)
    s = jnp.sum(jnp.exp(x - m), axis=0, keepdims=True)
    lse = m + jnp.log(s)
    cls = lax.broadcasted_iota(jnp.int32, x.shape, 0)
    oh = cls == t
    xt = jnp.sum(jnp.where(oh, x, 0.0), axis=0, keepdims=True)
    wt = jnp.sum(jnp.where(oh, w, 0.0), axis=0, keepdims=True)
    o_ref[0] = jnp.maximum(wt * (lse - xt), 0.0)


def _compute_loss(logits, targets, weights):
    B, C, H, W = logits.shape
    N = H * W
    chunk = 8192 if N % 8192 == 0 else N
    x3 = logits.reshape(B, C, N)
    t3 = targets.reshape(B, 1, N)
    w2 = jnp.broadcast_to(weights[:, None], (C, 128))
    loss = pl.pallas_call(
        _loss_body,
        grid=(B, N // chunk),
        in_specs=[
            pl.BlockSpec((C, 128), lambda b, c: (0, 0)),
            pl.BlockSpec((1, C, chunk), lambda b, c: (b, 0, c)),
            pl.BlockSpec((1, 1, chunk), lambda b, c: (b, 0, c)),
        ],
        out_specs=pl.BlockSpec((1, 1, chunk), lambda b, c: (b, 0, c)),
        out_shape=jax.ShapeDtypeStruct((B, 1, N), jnp.float32),
    )(w2, x3, t3)
    return loss.reshape(B * N)


def _make_hist_round(n, shift, first):
    per_tile = n // _NW
    schunk = min(4096, per_tile)
    nchunks = per_tile // schunk
    niter = schunk // _L
    mesh = plsc.VectorSubcoreMesh(core_axis_name="c", subcore_axis_name="s")

    @functools.partial(
        pl.kernel,
        out_type=(
            jax.ShapeDtypeStruct((_NW, 256, _L), jnp.int32),
            jax.ShapeDtypeStruct((_NW, 256, _L), jnp.float32),
        ),
        mesh=mesh,
        compiler_params=pltpu.CompilerParams(needs_layout_passes=False),
        scratch_types=[
            pltpu.VMEM((schunk,), jnp.float32),
            pltpu.VMEM((256, _L), jnp.int32),
            pltpu.VMEM((256, _L), jnp.float32),
            pltpu.VMEM((_L,), jnp.int32),
        ],
    )
    def round_kernel(loss_hbm, pref_hbm, cnt_hbm, sum_hbm,
                     data_v, cnt_v, sum_v, pref_v):
        cid = lax.axis_index("c")
        sid = lax.axis_index("s")
        wid = sid * _NC + cid
        base = wid * per_tile
        pltpu.sync_copy(pref_hbm, pref_v)
        prefu = plsc.bitcast(pref_v[...], jnp.uint32)
        zeros_i = jnp.zeros((_L,), jnp.int32)
        zeros_f = jnp.zeros((_L,), jnp.float32)
        ones_i = jnp.ones((_L,), jnp.int32)
        lanes = lax.iota(jnp.int32, _L)

        def zinit(b, carry):
            cnt_v[b] = zeros_i
            sum_v[b] = zeros_f
            return carry

        lax.fori_loop(0, 256, zinit, 0)

        def chunk_body(c, carry):
            pltpu.sync_copy(loss_hbm.at[pl.ds(base + c * schunk, schunk)],
                            data_v)

            def inner(i, icarry):
                v = data_v[pl.ds(i * _L, _L)]
                u = plsc.bitcast(v, jnp.uint32)
                dig = lax.shift_right_logical(u, jnp.uint32(shift))
                dig = dig & jnp.uint32(0xFF)
                bin_i = plsc.bitcast(dig, jnp.int32)
                if first:
                    pm = None
                else:
                    hi = lax.shift_right_logical(u, jnp.uint32(shift + 8))
                    pm = hi == prefu
                plsc.addupdate_scatter(cnt_v, [bin_i, lanes], ones_i, mask=pm)
                plsc.addupdate_scatter(sum_v, [bin_i, lanes], v, mask=pm)
                return icarry

            lax.fori_loop(0, niter, inner, 0)
            return carry

        lax.fori_loop(0, nchunks, chunk_body, 0)
        pltpu.sync_copy(cnt_v, cnt_hbm.at[wid])
        pltpu.sync_copy(sum_v, sum_hbm.at[wid])

    return round_kernel


def _hist_round(loss, pref_splat, shift, first):
    n = loss.shape[0]
    fn = _make_hist_round(n, shift, first)
    return fn(loss, pref_splat)


def kernel(logits, targets, weights):
    B, C, H, W = logits.shape
    n = B * H * W
    loss = _compute_loss(logits, targets, weights)
    if _MIN_KEPT >= n:
        return jnp.mean(loss)

    k = int(_THRESH * n)
    remaining = jnp.int32(k)
    s_gt = jnp.float32(0.0)
    prefix = jnp.int32(0)
    for r, shift in enumerate((24, 16, 8, 0)):
        pref_splat = jnp.broadcast_to(prefix, (_L,))
        cnt3, sum3 = _hist_round(loss, pref_splat, shift, r == 0)
        cnt = jnp.sum(cnt3, axis=(0, 2))
        sm = jnp.sum(sum3, axis=(0, 2))
        suf_c = jnp.cumsum(cnt[::-1])[::-1]
        suf_s = jnp.cumsum(sm[::-1])[::-1]
        b = jnp.sum((suf_c >= remaining).astype(jnp.int32)) - 1
        above_cnt = suf_c[b] - cnt[b]
        above_sum = suf_s[b] - sm[b]
        remaining = remaining - above_cnt
        s_gt = s_gt + above_sum
        prefix = (prefix << 8) | b
    tau = lax.bitcast_convert_type(prefix, jnp.float32)
    return (s_gt + remaining.astype(jnp.float32) * tau) / jnp.float32(k)


# speedup vs baseline: 35.0481x; 1.0712x over previous
import functools

import jax
import jax.numpy as jnp
from jax import lax
from jax.experimental import pallas as pl
from jax.experimental.pallas import tpu as pltpu
from jax.experimental.pallas import tpu_sc as plsc

_THRESH = 0.7
_MIN_KEPT = 100000

_NC = 2
_NS = 16
_L = 16
_NW = _NC * _NS
_NBINS = 256
_NGROUPS = _NBINS // _L
_UNROLL = 8


def _loss_body(w_ref, x_ref, t_ref, o_ref):
    x = x_ref[0]
    t = t_ref[0]
    w = w_ref[:, :1]
    m = jnp.max(x, axis=0, keepdims=True)
    s = jnp.sum(jnp.exp(x - m), axis=0, keepdims=True)
    lse = m + jnp.log(s)
    cls = lax.broadcasted_iota(jnp.int32, x.shape, 0)
    oh = cls == t
    xt = jnp.sum(jnp.where(oh, x, 0.0), axis=0, keepdims=True)
    wt = jnp.sum(jnp.where(oh, w, 0.0), axis=0, keepdims=True)
    o_ref[0] = jnp.maximum(wt * (lse - xt), 0.0)


def _compute_loss(logits, targets, weights):
    B, C, H, W = logits.shape
    N = H * W
    chunk = 8192 if N % 8192 == 0 else N
    x3 = logits.reshape(B, C, N)
    t3 = targets.reshape(B, 1, N)
    w2 = jnp.broadcast_to(weights[:, None], (C, 128))
    loss = pl.pallas_call(
        _loss_body,
        grid=(B, N // chunk),
        in_specs=[
            pl.BlockSpec((C, 128), lambda b, c: (0, 0)),
            pl.BlockSpec((1, C, chunk), lambda b, c: (b, 0, c)),
            pl.BlockSpec((1, 1, chunk), lambda b, c: (b, 0, c)),
        ],
        out_specs=pl.BlockSpec((1, 1, chunk), lambda b, c: (b, 0, c)),
        out_shape=jax.ShapeDtypeStruct((B, 1, N), jnp.float32),
    )(w2, x3, t3)
    return loss.reshape(B * N)


def _scan_histogram(cntp_v, prefix_prev, remaining_prev):
    lanes = lax.iota(jnp.int32, _L)
    acc = jnp.int32(0)
    chosen = jnp.int32(0)
    pre_sel = jnp.int32(0)
    for j in range(_NGROUPS):
        g = _NGROUPS - 1 - j
        row = jnp.zeros((_L,), jnp.int32)
        for t in range(_NW):
            row = row + cntp_v[t, pl.ds(g * _L, _L)]
        row_rev = lax.rev(row, (0,))
        suf = plsc.cumsum(row_rev)
        tot = suf[_L - 1]
        m = (acc + suf) >= remaining_prev
        pc = plsc.all_reduce_population_count(m)[0]
        f = _L - pc
        cand = g * _L + (_L - 1) - f
        above_in_g = jnp.sum(jnp.where(lanes < f, row_rev, 0))
        take = jnp.logical_and(acc < remaining_prev,
                               acc + tot >= remaining_prev)
        chosen = jnp.where(take, cand, chosen)
        pre_sel = jnp.where(take, acc + above_in_g, pre_sel)
        acc = acc + tot
    prefix = (prefix_prev << 8) | chosen
    remaining = remaining_prev - pre_sel
    return prefix, remaining


def _splat(x, dtype):
    return jnp.broadcast_to(x.astype(dtype), (_L,))


def _make_sel_kernel(n, k, stage):
    per_tile = n // _NW
    schunk = min(8192, per_tile)
    nchunks = per_tile // schunk
    shift = (24, 16, 8, 0, None)[stage]
    mesh = plsc.VectorSubcoreMesh(core_axis_name="c", subcore_axis_name="s")

    if stage == 4:
        out_type = (
            jax.ShapeDtypeStruct((_NW, _L), jnp.float32),
            jax.ShapeDtypeStruct((_L,), jnp.int32),
        )
    else:
        out_type = (
            jax.ShapeDtypeStruct((_NW, _NBINS), jnp.int32),
            jax.ShapeDtypeStruct((_L,), jnp.int32),
        )

    scratch = [
        pltpu.VMEM((schunk,), jnp.float32),
        pltpu.VMEM((schunk,), jnp.float32),
        pltpu.SemaphoreType.DMA,
        pltpu.SemaphoreType.DMA,
        pltpu.VMEM((_L,), jnp.int32),
        pltpu.VMEM((_L,), jnp.float32),
    ]
    if stage != 4:
        scratch += [
            pltpu.VMEM((_NBINS, _L), jnp.int32),
            pltpu.VMEM((_NBINS,), jnp.int32),
        ]
    if stage >= 1:
        scratch += [pltpu.VMEM((_NW, _NBINS), jnp.int32)]

    @functools.partial(
        pl.kernel,
        out_type=out_type,
        mesh=mesh,
        compiler_params=pltpu.CompilerParams(needs_layout_passes=False),
        scratch_types=scratch,
    )
    def sel_kernel(*refs):
        it = iter(refs)
        loss_hbm = next(it)
        cntp_hbm = next(it) if stage >= 1 else None
        statep_hbm = next(it) if stage >= 2 else None
        out0_hbm = next(it)
        state_hbm = next(it)
        buf0 = next(it)
        buf1 = next(it)
        sem0 = next(it)
        sem1 = next(it)
        stage_v = next(it)
        stagef_v = next(it)
        if stage != 4:
            hist_v = next(it)
            row_v = next(it)
        if stage >= 1:
            cntp_v = next(it)

        cid = lax.axis_index("c")
        sid = lax.axis_index("s")
        wid = sid * _NC + cid
        base = wid * per_tile
        lanes = lax.iota(jnp.int32, _L)

        if stage == 0:
            prefix = remaining = None
        else:
            pltpu.sync_copy(cntp_hbm, cntp_v)
            if stage == 1:
                prefix_prev = jnp.int32(0)
                remaining_prev = jnp.int32(k)
            else:
                pltpu.sync_copy(statep_hbm, stage_v)
                sv = stage_v[...]
                prefix_prev = sv[0]
                remaining_prev = sv[1]
            prefix, remaining = _scan_histogram(
                cntp_v, prefix_prev, remaining_prev)

        bufs = (buf0, buf1)
        sems = (sem0, sem1)
        copies = [None, None]
        copies[0] = pltpu.async_copy(
            loss_hbm.at[pl.ds(base, schunk)], bufs[0], sems[0])

        if stage != 4:
            zeros_row = jnp.zeros((_L,), jnp.int32)

            def zinit(b, carry):
                hist_v[b] = zeros_row
                return carry

            lax.fori_loop(0, _NBINS, zinit, 0)

            ones_i = jnp.ones((_L,), jnp.int32)
            if stage >= 1:
                pref_u = _splat(prefix, jnp.uint32)

            for c in range(nchunks):
                cur = c % 2
                nxt = (c + 1) % 2
                if c + 1 < nchunks:
                    copies[nxt] = pltpu.async_copy(
                        loss_hbm.at[pl.ds(base + (c + 1) * schunk, schunk)],
                        bufs[nxt], sems[nxt])
                copies[cur].wait()

                def body(i, carry):
                    for t in range(_UNROLL):
                        v = bufs[cur][pl.ds(i * (_L * _UNROLL) + t * _L, _L)]
                        u = plsc.bitcast(v, jnp.uint32)
                        dig = lax.shift_right_logical(u, jnp.uint32(shift))
                        dig = dig & jnp.uint32(0xFF)
                        bin_i = plsc.bitcast(dig, jnp.int32)
                        if stage == 0:
                            pm = None
                        else:
                            hi = lax.shift_right_logical(
                                u, jnp.uint32(shift + 8))
                            pm = hi == pref_u
                        plsc.addupdate_scatter(
                            hist_v, [bin_i, lanes], ones_i, mask=pm)
                    return carry

                lax.fori_loop(0, schunk // (_L * _UNROLL), body, 0)

            for g in range(_NGROUPS):
                c16 = jnp.zeros((_L,), jnp.int32)
                for j in range(_L):
                    s = jnp.sum(hist_v[g * _L + j])
                    c16 = jnp.where(lanes == j, s, c16)
                row_v[pl.ds(g * _L, _L)] = c16
            pltpu.sync_copy(row_v, out0_hbm.at[wid])

            @pl.when(wid == 0)
            def _wstate():
                if stage >= 1:
                    vec = jnp.where(lanes == 0, prefix,
                                    jnp.where(lanes == 1, remaining, 0))
                else:
                    vec = jnp.zeros((_L,), jnp.int32)
                stage_v[...] = vec
                pltpu.sync_copy(stage_v, state_hbm)
        else:
            tau_u = _splat(prefix, jnp.uint32)
            acc_total = jnp.zeros((_L,), jnp.float32)
            for c in range(nchunks):
                cur = c % 2
                nxt = (c + 1) % 2
                if c + 1 < nchunks:
                    copies[nxt] = pltpu.async_copy(
                        loss_hbm.at[pl.ds(base + (c + 1) * schunk, schunk)],
                        bufs[nxt], sems[nxt])
                copies[cur].wait()

                def body(i, acc):
                    for t in range(_UNROLL):
                        v = bufs[cur][pl.ds(i * (_L * _UNROLL) + t * _L, _L)]
                        u = plsc.bitcast(v, jnp.uint32)
                        acc = acc + jnp.where(u > tau_u, v, 0.0)
                    return acc

                acc_total = lax.fori_loop(
                    0, schunk // (_L * _UNROLL), body, acc_total)

            stagef_v[...] = acc_total
            pltpu.sync_copy(stagef_v, out0_hbm.at[wid])

            @pl.when(wid == 0)
            def _wstate4():
                vec = jnp.where(lanes == 0, prefix,
                                jnp.where(lanes == 1, remaining, 0))
                stage_v[...] = vec
                pltpu.sync_copy(stage_v, state_hbm)

    return sel_kernel


def _select_topk_mean(loss, k):
    n = loss.shape[0]
    cnt0, _ = _make_sel_kernel(n, k, 0)(loss)
    cnt1, st1 = _make_sel_kernel(n, k, 1)(loss, cnt0)
    cnt2, st2 = _make_sel_kernel(n, k, 2)(loss, cnt1, st1)
    cnt3, st3 = _make_sel_kernel(n, k, 3)(loss, cnt2, st2)
    partials, st4 = _make_sel_kernel(n, k, 4)(loss, cnt3, st3)
    tau = lax.bitcast_convert_type(st4[0], jnp.float32)
    remaining = st4[1].astype(jnp.float32)
    return (jnp.sum(partials) + remaining * tau) / jnp.float32(k)


def kernel(logits, targets, weights):
    B, C, H, W = logits.shape
    n = B * H * W
    loss = _compute_loss(logits, targets, weights)
    if _MIN_KEPT >= n:
        return jnp.mean(loss)
    k = int(_THRESH * n)
    return _select_topk_mean(loss, k)
